# baseline (speedup 1.0000x reference)
import functools

import jax
import jax.numpy as jnp
from jax import lax
from jax.experimental import pallas as pl
from jax.experimental.pallas import tpu as pltpu
from jax.experimental.pallas import tpu_sc as plsc

LANES = 16
NWORKERS = 32
NBUF = 4
CHUNK = 128


def _sc_pool(enc_flat, bsz, seq, emb_table, cat0, cat1, cat2, add_flat):
  dim = emb_table.shape[1]
  bpw = bsz // NWORKERS
  ngrp = dim // LANES
  mesh = plsc.VectorSubcoreMesh(core_axis_name="c", subcore_axis_name="s")

  @functools.partial(
      pl.kernel,
      out_type=(
          jax.ShapeDtypeStruct((bsz, dim), jnp.float32),
          jax.ShapeDtypeStruct((bsz, dim), jnp.float32),
      ),
      mesh=mesh,
      scratch_types=[
          pltpu.VMEM((NBUF, 2 * CHUNK), jnp.int32),
          pltpu.VMEM((NBUF, seq, dim), jnp.float32),
          pltpu.VMEM((bpw, dim), jnp.float32),
          pltpu.VMEM((3, bpw), jnp.int32),
          pltpu.VMEM((bpw, dim), jnp.float32),
          pltpu.VMEM((bpw, dim), jnp.float32),
          pltpu.VMEM((bpw, dim), jnp.float32),
          pltpu.SemaphoreType.DMA,
          pltpu.SemaphoreType.DMA,
          pltpu.SemaphoreType.DMA,
          pltpu.SemaphoreType.DMA,
      ],
      compiler_params=pltpu.CompilerParams(use_tc_tiling_on_sc=False),
  )
  def k(enc_hbm, emb_hbm, c0_hbm, c1_hbm, c2_hbm, addt_hbm,
        sums_hbm, cats_hbm,
        idx_v, rows_v, acc_v, cidx_v, ca_v, cb_v, cc_v,
        sem0, sem1, sem2, sem3):
    sems = (sem0, sem1, sem2, sem3)
    wid = lax.axis_index("s") * 2 + lax.axis_index("c")
    base = wid * bpw

    def gather_descs(buf):
      return (
          pltpu.make_async_copy(
              emb_hbm.at[idx_v.at[buf, pl.ds(0, CHUNK)]],
              rows_v.at[buf, pl.ds(0, CHUNK)], sems[buf]),
          pltpu.make_async_copy(
              emb_hbm.at[idx_v.at[buf, pl.ds(CHUNK, seq - CHUNK)]],
              rows_v.at[buf, pl.ds(CHUNK, seq - CHUNK)], sems[buf]),
      )

    def issue(buf, row):
      pltpu.sync_copy(enc_hbm.at[pl.ds(row * seq, CHUNK)],
                      idx_v.at[buf, pl.ds(0, CHUNK)])
      pltpu.sync_copy(enc_hbm.at[pl.ds(row * seq + CHUNK, CHUNK)],
                      idx_v.at[buf, pl.ds(CHUNK, CHUNK)])
      for d in gather_descs(buf):
        d.start()

    for buf in range(NBUF):
      issue(buf, base + buf)

    for j, (tab, dst) in enumerate(
        ((c0_hbm, ca_v), (c1_hbm, cb_v), (c2_hbm, cc_v))):
      pltpu.sync_copy(addt_hbm.at[pl.ds(j * bsz + base, bpw)], cidx_v.at[j])
      pltpu.sync_copy(tab.at[cidx_v.at[j]], dst)

    @pl.loop(0, bpw, unroll=4)
    def _(b):
      for j in range(ngrp):
        sl = pl.ds(j * LANES, LANES)
        ca_v[b, sl] = ca_v[b, sl] + cb_v[b, sl] + cc_v[b, sl]

    pltpu.sync_copy(ca_v, cats_hbm.at[pl.ds(base, bpw)])

    @pl.loop(0, bpw // NBUF)
    def _(i):
      for buf in range(NBUF):
        b_local = i * NBUF + buf
        for d in gather_descs(buf):
          d.wait()
        zeros = (jnp.zeros((LANES,), jnp.float32),) * ngrp
        @pl.loop(0, seq, init_carry=zeros, unroll=8)
        def totals(t, carry):
          return tuple(
              c + rows_v[buf, t, pl.ds(j * LANES, LANES)]
              for j, c in enumerate(carry))
        for j in range(ngrp):
          acc_v[b_local, pl.ds(j * LANES, LANES)] = totals[j]
        @pl.when(i < bpw // NBUF - 1)
        def _():
          issue(buf, base + b_local + NBUF)

    pltpu.sync_copy(acc_v, sums_hbm.at[pl.ds(base, bpw)])

  return k(enc_flat, emb_table, cat0, cat1, cat2, add_flat)


def _tc_head(sums, cats, encoded_text, w_t, bias):
  bsz, seq = encoded_text.shape
  dim = sums.shape[1]
  ncls = w_t.shape[1]
  blk = 256

  def body(sums_ref, cats_ref, enc_ref, wt_ref, b_ref, out_ref):
    cnt = jnp.sum((enc_ref[...] != 0).astype(jnp.float32), axis=1,
                  keepdims=True)
    x = jnp.where(cnt > 0.0, sums_ref[...] / cnt, 0.0)
    x = x + cats_ref[...]
    z = lax.dot_general(x, wt_ref[...], (((1,), (0,)), ((), ())),
                        preferred_element_type=jnp.float32)
    out_ref[...] = z + b_ref[...]

  return pl.pallas_call(
      body,
      grid=(bsz // blk,),
      in_specs=[
          pl.BlockSpec((blk, dim), lambda i: (i, 0)),
          pl.BlockSpec((blk, dim), lambda i: (i, 0)),
          pl.BlockSpec((blk, seq), lambda i: (i, 0)),
          pl.BlockSpec((dim, ncls), lambda i: (0, 0)),
          pl.BlockSpec((1, ncls), lambda i: (0, 0)),
      ],
      out_specs=pl.BlockSpec((blk, ncls), lambda i: (i, 0)),
      out_shape=jax.ShapeDtypeStruct((bsz, ncls), jnp.float32),
  )(sums, cats, encoded_text, w_t, bias)


def kernel(encoded_text, additional_inputs, emb_table, cat0, cat1, cat2, W, b):
  bsz, seq = encoded_text.shape
  enc_flat = jnp.pad(encoded_text.reshape(-1), (0, 2 * CHUNK - seq % (2 * CHUNK)))
  sums, cats = _sc_pool(enc_flat, bsz, seq, emb_table,
                        cat0, cat1, cat2, additional_inputs.T.reshape(-1))
  return _tc_head(sums, cats, encoded_text, W.T, b.reshape(1, -1))
